# Initial kernel scaffold; baseline (speedup 1.0000x reference)
import jax
import jax.numpy as jnp
from jax.experimental import pallas as pl


def kernel(x, t, W, b):
    raise NotImplementedError("write your pallas kernel here")


# speedup vs baseline: 34.7564x; 34.7564x over previous
import jax
import jax.numpy as jnp
from jax.experimental import pallas as pl


def _body(x_ref, t_ref, w_ref, b_ref, o_ref, *, bn, ec, c):
    acc = jnp.dot(x_ref[...], w_ref[...], preferred_element_type=jnp.float32)
    acc = acc + b_ref[...]
    lane_e = jax.lax.broadcasted_iota(jnp.int32, (bn, ec), 1) // c
    masked = jnp.where(lane_e == t_ref[...], acc, 0.0)
    jrow = jax.lax.broadcasted_iota(jnp.int32, (ec, c), 0) % c
    ccol = jax.lax.broadcasted_iota(jnp.int32, (ec, c), 1)
    fold = (jrow == ccol).astype(jnp.float32)
    o_ref[...] = jnp.dot(masked, fold, preferred_element_type=jnp.float32)


def kernel(x, t, W, b):
    n, d = x.shape
    e, _, c = W.shape
    ec = e * c
    w_all = W.transpose(1, 0, 2).reshape(d, ec)
    b_all = b.reshape(1, ec)
    t32 = t.astype(jnp.int32).reshape(n, 1)
    bn = 1024
    import functools
    body = functools.partial(_body, bn=bn, ec=ec, c=c)
    return pl.pallas_call(
        body,
        grid=(n // bn,),
        in_specs=[
            pl.BlockSpec((bn, d), lambda i: (i, 0)),
            pl.BlockSpec((bn, 1), lambda i: (i, 0)),
            pl.BlockSpec((d, ec), lambda i: (0, 0)),
            pl.BlockSpec((1, ec), lambda i: (0, 0)),
        ],
        out_specs=pl.BlockSpec((bn, c), lambda i: (i, 0)),
        out_shape=jax.ShapeDtypeStruct((n, c), jnp.float32),
    )(x, t32, w_all, b_all)


# speedup vs baseline: 37.3093x; 1.0734x over previous
import jax
import jax.numpy as jnp
from jax.experimental import pallas as pl


def _body(x_ref, t_ref, w_ref, b_ref, o_ref, *, bn, ec, c):
    acc = jnp.dot(x_ref[...], w_ref[...], preferred_element_type=jnp.float32)
    acc = acc + b_ref[...]
    lane_e = jax.lax.broadcasted_iota(jnp.int32, (bn, ec), 1) // c
    masked = jnp.where(lane_e == t_ref[...], acc, 0.0)
    jrow = jax.lax.broadcasted_iota(jnp.int32, (ec, c), 0) % c
    ccol = jax.lax.broadcasted_iota(jnp.int32, (ec, c), 1)
    fold = (jrow == ccol).astype(jnp.float32)
    o_ref[...] = jnp.dot(masked, fold, preferred_element_type=jnp.float32)


def kernel(x, t, W, b):
    n, d = x.shape
    e, _, c = W.shape
    ec = e * c
    w_all = W.transpose(1, 0, 2).reshape(d, ec)
    b_all = b.reshape(1, ec)
    t32 = t.astype(jnp.int32).reshape(n, 1)
    bn = 2048
    import functools
    body = functools.partial(_body, bn=bn, ec=ec, c=c)
    return pl.pallas_call(
        body,
        grid=(n // bn,),
        in_specs=[
            pl.BlockSpec((bn, d), lambda i: (i, 0)),
            pl.BlockSpec((bn, 1), lambda i: (i, 0)),
            pl.BlockSpec((d, ec), lambda i: (0, 0)),
            pl.BlockSpec((1, ec), lambda i: (0, 0)),
        ],
        out_specs=pl.BlockSpec((bn, c), lambda i: (i, 0)),
        out_shape=jax.ShapeDtypeStruct((n, c), jnp.float32),
    )(x, t32, w_all, b_all)
